# Initial kernel scaffold; baseline (speedup 1.0000x reference)
import jax
import jax.numpy as jnp
from jax.experimental import pallas as pl


def kernel(x, w1, b1, w2, b2, l1, lb1, l2, lb2):
    raise NotImplementedError("write your pallas kernel here")


# speedup vs baseline: 3.5806x; 3.5806x over previous
import numpy as np
import jax
import jax.numpy as jnp
from jax.experimental import pallas as pl
from jax.experimental.pallas import tpu as pltpu

_BB = 16
_T = 7
_KB = 168
_MB = 512


def _band_rows() -> np.ndarray:
    R = np.full((_KB, _T), 75, dtype=np.int32)
    for i in range(5):
        for u in range(11):
            for c in range(3):
                k = i * 33 + 3 * u + c
                for owp in range(_T):
                    j = u - owp
                    if 0 <= j < 5:
                        R[k, owp] = (i * 5 + j) * 3 + c
    return R


_R = _band_rows()


def _conv_kernel(x_ref, wb_ref, b1_ref, w2_ref, b2_ref, o_ref):
    X = x_ref[...]
    groups = []
    zpad = jnp.zeros((_BB, 28, 3), jnp.bfloat16)
    for t in range(4):
        parts = [X[:, i:i + 28, 21 * t:21 * t + 33] for i in range(5)]
        parts.append(zpad)
        groups.append(jnp.concatenate(parts, axis=-1))
    Xc = jnp.stack(groups, axis=2)
    Y = jnp.dot(Xc.reshape(_BB * 112, _KB), wb_ref[...],
                preferred_element_type=jnp.float32)
    Y = Y.reshape(_BB, 28, 4, 7, 64) + b1_ref[...][0]
    Y = jnp.maximum(Y, 0.0).reshape(_BB, 28, 28, 64)
    Y = Y.reshape(_BB, 14, 2, 28, 64).max(axis=2)
    P1 = Y.reshape(_BB, 14, 14, 2, 64).max(axis=3)
    P1 = P1.astype(jnp.bfloat16)

    cols = []
    for i in range(5):
        Ai = P1[:, i:i + 10, :, :]
        for j in range(5):
            cols.append(Ai[:, :, j:j + 10, :])
    P = jnp.concatenate(cols, axis=-1).reshape(_BB * 100, 1600)
    Z = jnp.dot(P, w2_ref[...], preferred_element_type=jnp.float32)
    Z = jnp.maximum(Z + b2_ref[...], 0.0)
    Z = Z.reshape(_BB, 5, 2, 5, 2, 64).max(axis=(2, 4))
    o_ref[...] = Z.reshape(_BB, 25, 64).astype(jnp.bfloat16)


def _mlp_kernel(x_ref, l1_ref, b1_ref, l2_ref, b2_ref, o_ref):
    h = jnp.dot(x_ref[...], l1_ref[...], preferred_element_type=jnp.float32)
    h = jnp.maximum(h + b1_ref[...], 0.0).astype(jnp.bfloat16)
    o = jnp.dot(h, l2_ref[...], preferred_element_type=jnp.float32)
    o_ref[...] = jnp.maximum(o + b2_ref[...], 0.0)


def kernel(x, w1, b1, w2, b2, l1, lb1, l2, lb2):
    B = x.shape[0]
    xb = jnp.transpose(x, (0, 2, 3, 1)).reshape(B, 32, 96).astype(jnp.bfloat16)
    wb = w1[jnp.asarray(_R)].reshape(_KB, _T * 64).astype(jnp.bfloat16)
    w2b = w2.astype(jnp.bfloat16)
    feats = pl.pallas_call(
        _conv_kernel,
        out_shape=jax.ShapeDtypeStruct((B, 25, 64), jnp.bfloat16),
        grid=(B // _BB,),
        in_specs=[
            pl.BlockSpec((_BB, 32, 96), lambda b: (b, 0, 0)),
            pl.BlockSpec((_KB, _T * 64), lambda b: (0, 0)),
            pl.BlockSpec((1, 64), lambda b: (0, 0)),
            pl.BlockSpec((1600, 64), lambda b: (0, 0)),
            pl.BlockSpec((1, 64), lambda b: (0, 0)),
        ],
        out_specs=pl.BlockSpec((_BB, 25, 64), lambda b: (b, 0, 0)),
        compiler_params=pltpu.CompilerParams(
            dimension_semantics=("parallel",)),
    )(xb, wb, b1, w2b, b2)
    feats = feats.reshape(B, 1600)
    mb = B if B < _MB else _MB
    return pl.pallas_call(
        _mlp_kernel,
        out_shape=jax.ShapeDtypeStruct((B, 192), jnp.float32),
        grid=(B // mb,),
        in_specs=[
            pl.BlockSpec((mb, 1600), lambda i: (i, 0)),
            pl.BlockSpec((1600, 384), lambda i: (0, 0)),
            pl.BlockSpec((1, 384), lambda i: (0, 0)),
            pl.BlockSpec((384, 192), lambda i: (0, 0)),
            pl.BlockSpec((1, 192), lambda i: (0, 0)),
        ],
        out_specs=pl.BlockSpec((mb, 192), lambda i: (i, 0)),
        compiler_params=pltpu.CompilerParams(
            dimension_semantics=("parallel",)),
    )(feats, l1.astype(jnp.bfloat16), lb1, l2.astype(jnp.bfloat16), lb2)


# speedup vs baseline: 5.7108x; 1.5949x over previous
import numpy as np
import jax
import jax.numpy as jnp
from jax.experimental import pallas as pl
from jax.experimental.pallas import tpu as pltpu

_BB = 16
_T = 4
_KB = 128
_MB = 512


def _band_rows() -> np.ndarray:
    R = np.full((_KB, _T), 75, dtype=np.int32)
    for i in range(5):
        for u in range(8):
            for c in range(3):
                k = i * 24 + 3 * u + c
                for owp in range(_T):
                    j = u - owp
                    if 0 <= j < 5:
                        R[k, owp] = (i * 5 + j) * 3 + c
    return R


_R = _band_rows()


def _conv_kernel(x_ref, wb_ref, b1_ref, w2_ref, b2_ref, o_ref):
    X = x_ref[...]
    groups = []
    zpad = jnp.zeros((_BB, 28, 8), jnp.bfloat16)
    for t in range(7):
        parts = [X[:, i:i + 28, 12 * t:12 * t + 24] for i in range(5)]
        parts.append(zpad)
        groups.append(jnp.concatenate(parts, axis=-1))
    Xc = jnp.stack(groups, axis=2)
    Y = jnp.dot(Xc.reshape(_BB * 196, _KB), wb_ref[...],
                preferred_element_type=jnp.float32)
    Y = jnp.maximum(Y + b1_ref[...], 0.0).astype(jnp.bfloat16)
    Yw = jnp.concatenate(
        [jnp.maximum(Y[:, 0:64], Y[:, 64:128]),
         jnp.maximum(Y[:, 128:192], Y[:, 192:256])], axis=-1)
    Yh = Yw.reshape(_BB, 14, 2, 7, 128).max(axis=2)
    P1 = jnp.stack([Yh[..., 0:64], Yh[..., 64:128]],
                   axis=3).reshape(_BB, 14, 14, 64)

    cols = []
    for i in range(5):
        Ai = P1[:, i:i + 10, :, :]
        for j in range(5):
            cols.append(Ai[:, :, j:j + 10, :])
    P = jnp.concatenate(cols, axis=-1).reshape(_BB * 100, 1600)
    Z = jnp.dot(P, w2_ref[...], preferred_element_type=jnp.float32)
    Z = jnp.maximum(Z + b2_ref[...], 0.0)
    Z = Z.reshape(_BB, 5, 2, 5, 2, 64).max(axis=(2, 4))
    o_ref[...] = Z.reshape(_BB, 25, 64).astype(jnp.bfloat16)


def _mlp_kernel(x_ref, l1_ref, b1_ref, l2_ref, b2_ref, o_ref):
    h = jnp.dot(x_ref[...], l1_ref[...], preferred_element_type=jnp.float32)
    h = jnp.maximum(h + b1_ref[...], 0.0).astype(jnp.bfloat16)
    o = jnp.dot(h, l2_ref[...], preferred_element_type=jnp.float32)
    o_ref[...] = jnp.maximum(o + b2_ref[...], 0.0)


def kernel(x, w1, b1, w2, b2, l1, lb1, l2, lb2):
    B = x.shape[0]
    xb = jnp.transpose(x, (0, 2, 3, 1)).reshape(B, 32, 96).astype(jnp.bfloat16)
    wb = w1[jnp.asarray(_R)].reshape(_KB, _T * 64).astype(jnp.bfloat16)
    b1t = jnp.tile(b1, (1, _T))
    w2b = w2.astype(jnp.bfloat16)
    feats = pl.pallas_call(
        _conv_kernel,
        out_shape=jax.ShapeDtypeStruct((B, 25, 64), jnp.bfloat16),
        grid=(B // _BB,),
        in_specs=[
            pl.BlockSpec((_BB, 32, 96), lambda b: (b, 0, 0)),
            pl.BlockSpec((_KB, _T * 64), lambda b: (0, 0)),
            pl.BlockSpec((1, _T * 64), lambda b: (0, 0)),
            pl.BlockSpec((1600, 64), lambda b: (0, 0)),
            pl.BlockSpec((1, 64), lambda b: (0, 0)),
        ],
        out_specs=pl.BlockSpec((_BB, 25, 64), lambda b: (b, 0, 0)),
        compiler_params=pltpu.CompilerParams(
            dimension_semantics=("parallel",)),
    )(xb, wb, b1t, w2b, b2)
    feats = feats.reshape(B, 1600)
    mb = B if B < _MB else _MB
    return pl.pallas_call(
        _mlp_kernel,
        out_shape=jax.ShapeDtypeStruct((B, 192), jnp.float32),
        grid=(B // mb,),
        in_specs=[
            pl.BlockSpec((mb, 1600), lambda i: (i, 0)),
            pl.BlockSpec((1600, 384), lambda i: (0, 0)),
            pl.BlockSpec((1, 384), lambda i: (0, 0)),
            pl.BlockSpec((384, 192), lambda i: (0, 0)),
            pl.BlockSpec((1, 192), lambda i: (0, 0)),
        ],
        out_specs=pl.BlockSpec((mb, 192), lambda i: (i, 0)),
        compiler_params=pltpu.CompilerParams(
            dimension_semantics=("parallel",)),
    )(feats, l1.astype(jnp.bfloat16), lb1, l2.astype(jnp.bfloat16), lb2)


# speedup vs baseline: 16.8322x; 2.9474x over previous
import numpy as np
import jax
import jax.numpy as jnp
from jax.experimental import pallas as pl
from jax.experimental.pallas import tpu as pltpu

_BB = 32
_MB = 512


def _conv1_band() -> np.ndarray:
    R = np.full((640, 28), 75, dtype=np.int32)
    for i in range(5):
        for w in range(32):
            for c4 in range(3):
                k = i * 128 + w * 4 + c4
                for dx in range(2):
                    for kk in range(14):
                        j = w - (2 * kk + dx)
                        if 0 <= j < 5:
                            R[k, dx * 14 + kk] = (i * 5 + j) * 3 + c4
    return R


def _conv2_band() -> np.ndarray:
    R = np.full((4480, 10), 1600, dtype=np.int32)
    for i2 in range(5):
        for w in range(14):
            for c in range(64):
                k = i2 * 896 + w * 64 + c
                for fx in range(2):
                    for qw in range(5):
                        j2 = w - (2 * qw + fx)
                        if 0 <= j2 < 5:
                            R[k, fx * 5 + qw] = (i2 * 5 + j2) * 64 + c
    return R


_R1 = _conv1_band()
_R2 = _conv2_band()


def _conv_kernel(x_ref, w1_ref, b1_ref, w2_ref, b2_ref, o_ref):
    bb = x_ref.shape[1]
    X = x_ref[...]
    Xc = jnp.concatenate(
        [X[i:i + 28].reshape(28 * bb, 128) for i in range(5)], axis=-1)
    Y = jnp.dot(Xc, w1_ref[...],
                preferred_element_type=jnp.float32)
    Y = jnp.maximum(Y[:, 0:896], Y[:, 896:1792])
    Y = jnp.maximum(Y + b1_ref[...], 0.0).astype(jnp.bfloat16)
    P1 = Y.reshape(14, 2, bb, 896).max(axis=1)
    P1 = P1.reshape(14 * bb, 896)

    Pc = jnp.concatenate(
        [P1[i * bb:(i + 10) * bb] for i in range(5)], axis=-1)
    Z = jnp.dot(Pc, w2_ref[...],
                preferred_element_type=jnp.float32)
    Z = jnp.maximum(Z[:, 0:320], Z[:, 320:640])
    Z = jnp.maximum(Z + b2_ref[...], 0.0)
    Zp = Z.reshape(5, 2, bb, 320).max(axis=1)
    o_ref[...] = jnp.concatenate(
        [Zp[q] for q in range(5)], axis=-1).astype(jnp.bfloat16)


def _mlp_kernel(x_ref, l1_ref, b1_ref, l2_ref, b2_ref, o_ref):
    h = jnp.dot(x_ref[...], l1_ref[...], preferred_element_type=jnp.float32)
    h = jnp.maximum(h + b1_ref[...], 0.0).astype(jnp.bfloat16)
    o = jnp.dot(h, l2_ref[...], preferred_element_type=jnp.float32)
    o_ref[...] = jnp.maximum(o + b2_ref[...], 0.0)


def kernel(x, w1, b1, w2, b2, l1, lb1, l2, lb2):
    B = x.shape[0]
    bb = B if B < _BB else _BB
    xq = jnp.pad(jnp.transpose(x, (2, 0, 3, 1)), ((0, 0), (0, 0), (0, 0), (0, 1)))
    xq = xq.reshape(32, B, 128).astype(jnp.bfloat16)
    w1b = w1[jnp.asarray(_R1)].reshape(640, 28 * 64).astype(jnp.bfloat16)
    w2e = jnp.concatenate([w2, jnp.zeros((1, 64), w2.dtype)], axis=0)
    w2b = w2e[jnp.asarray(_R2)].reshape(4480, 640).astype(jnp.bfloat16)
    b1t = jnp.tile(b1, (1, 14))
    b2t = jnp.tile(b2, (1, 5))
    feats = pl.pallas_call(
        _conv_kernel,
        out_shape=jax.ShapeDtypeStruct((B, 1600), jnp.bfloat16),
        grid=(B // bb,),
        in_specs=[
            pl.BlockSpec((32, bb, 128), lambda b: (0, b, 0)),
            pl.BlockSpec((640, 28 * 64), lambda b: (0, 0)),
            pl.BlockSpec((1, 896), lambda b: (0, 0)),
            pl.BlockSpec((4480, 640), lambda b: (0, 0)),
            pl.BlockSpec((1, 320), lambda b: (0, 0)),
        ],
        out_specs=pl.BlockSpec((bb, 1600), lambda b: (b, 0)),
        compiler_params=pltpu.CompilerParams(
            dimension_semantics=("parallel",)),
    )(xq, w1b, b1t, w2b, b2t)
    mb = B if B < _MB else _MB
    return pl.pallas_call(
        _mlp_kernel,
        out_shape=jax.ShapeDtypeStruct((B, 192), jnp.float32),
        grid=(B // mb,),
        in_specs=[
            pl.BlockSpec((mb, 1600), lambda i: (i, 0)),
            pl.BlockSpec((1600, 384), lambda i: (0, 0)),
            pl.BlockSpec((1, 384), lambda i: (0, 0)),
            pl.BlockSpec((384, 192), lambda i: (0, 0)),
            pl.BlockSpec((1, 192), lambda i: (0, 0)),
        ],
        out_specs=pl.BlockSpec((mb, 192), lambda i: (i, 0)),
        compiler_params=pltpu.CompilerParams(
            dimension_semantics=("parallel",)),
    )(feats, l1.astype(jnp.bfloat16), lb1, l2.astype(jnp.bfloat16), lb2)
